# Initial kernel scaffold; baseline (speedup 1.0000x reference)
import functools

import jax
import jax.numpy as jnp
from jax import lax
from jax.experimental import pallas as pl
from jax.experimental.pallas import tpu as pltpu
from jax.experimental.pallas import tpu_sc as plsc

N_EMBD = 64
CODEBOOK = 100000
G = 8

NC, NS = 2, 16
NW = NC * NS
CHUNK = 128


def kernel(idx, tables):
    B, T, g_dim = idx.shape
    N = B * T
    per_w = N // NW
    n_chunks = per_w // CHUNK

    offs = jnp.arange(G, dtype=jnp.int32) * CODEBOOK
    idx_flat = idx.reshape(N, G).astype(jnp.int32) + offs
    idx_r = idx_flat.reshape(NW * n_chunks, CHUNK, G).transpose(0, 2, 1)
    table_flat = tables.reshape(G * CODEBOOK, N_EMBD)

    mesh = plsc.VectorSubcoreMesh(core_axis_name="c", subcore_axis_name="s")

    @functools.partial(
        pl.kernel,
        out_type=jax.ShapeDtypeStruct((N, N_EMBD), jnp.float32),
        mesh=mesh,
        scratch_types=[
            pltpu.VMEM((G, CHUNK), jnp.int32),
            pltpu.VMEM((CHUNK, N_EMBD), jnp.float32),
            pltpu.SemaphoreType.DMA,
        ],
    )
    def body(idx_hbm, tab_hbm, out_hbm, idx_v, acc_v, sem):
        wid = lax.axis_index("s") * NC + lax.axis_index("c")

        def chunk_body(j, _):
            pltpu.sync_copy(idx_hbm.at[wid * n_chunks + j], idx_v)
            pltpu.async_copy(tab_hbm.at[idx_v.at[0]], acc_v, sem).wait()
            for g in range(1, G):
                pltpu.async_copy(tab_hbm.at[idx_v.at[g]], acc_v, sem,
                                 add=True).wait()
            base = (wid * n_chunks + j) * CHUNK
            pltpu.sync_copy(acc_v, out_hbm.at[pl.ds(base, CHUNK)])
            return ()

        lax.fori_loop(0, n_chunks, chunk_body, ())

    out = body(idx_r, table_flat)
    return out.reshape(B, T, N_EMBD)


# speedup vs baseline: 10.2899x; 10.2899x over previous
import functools

import jax
import jax.numpy as jnp
from jax import lax
from jax.experimental import pallas as pl
from jax.experimental.pallas import tpu as pltpu
from jax.experimental.pallas import tpu_sc as plsc

N_EMBD = 64
CODEBOOK = 100000
G = 8

NC, NS = 2, 16
NW = NC * NS
CHUNK = 128


def kernel(idx, tables):
    B, T, g_dim = idx.shape
    N = B * T
    per_w = N // NW
    n_chunks = per_w // CHUNK

    offs = jnp.arange(G, dtype=jnp.int32) * CODEBOOK
    idx_flat = idx.reshape(N, G).astype(jnp.int32) + offs
    idx_r = idx_flat.reshape(NW * n_chunks, CHUNK, G).transpose(0, 2, 1)
    table_flat = tables.reshape(G * CODEBOOK, N_EMBD)

    mesh = plsc.VectorSubcoreMesh(core_axis_name="c", subcore_axis_name="s")

    @functools.partial(
        pl.kernel,
        out_type=jax.ShapeDtypeStruct((N, N_EMBD), jnp.float32),
        mesh=mesh,
        compiler_params=pltpu.CompilerParams(use_tc_tiling_on_sc=False),
        scratch_types=[
            pltpu.VMEM((G, CHUNK), jnp.int32),
            pltpu.VMEM((CHUNK, N_EMBD), jnp.float32),
            pltpu.SemaphoreType.DMA,
        ],
    )
    def body(idx_hbm, tab_hbm, out_hbm, idx_v, acc_v, sem):
        wid = lax.axis_index("s") * NC + lax.axis_index("c")

        def chunk_body(j, _):
            pltpu.sync_copy(idx_hbm.at[wid * n_chunks + j], idx_v)
            pltpu.async_copy(tab_hbm.at[idx_v.at[0]], acc_v, sem).wait()
            for g in range(1, G):
                pltpu.async_copy(tab_hbm.at[idx_v.at[g]], acc_v, sem,
                                 add=True).wait()
            base = (wid * n_chunks + j) * CHUNK
            pltpu.sync_copy(acc_v, out_hbm.at[pl.ds(base, CHUNK)])
            return ()

        lax.fori_loop(0, n_chunks, chunk_body, ())

    out = body(idx_r, table_flat)
    return out.reshape(B, T, N_EMBD)


# speedup vs baseline: 13.7123x; 1.3326x over previous
import functools

import jax
import jax.numpy as jnp
from jax import lax
from jax.experimental import pallas as pl
from jax.experimental.pallas import tpu as pltpu
from jax.experimental.pallas import tpu_sc as plsc

N_EMBD = 64
CODEBOOK = 100000
G = 8

NC, NS = 2, 16
NW = NC * NS
CHUNK = 128
NBUF = 2


def kernel(idx, tables):
    B, T, g_dim = idx.shape
    N = B * T
    per_w = N // NW
    n_chunks = per_w // CHUNK

    offs = jnp.arange(G, dtype=jnp.int32) * CODEBOOK
    idx_flat = idx.reshape(N, G).astype(jnp.int32) + offs
    idx_r = idx_flat.reshape(NW * n_chunks, CHUNK, G).transpose(0, 2, 1)
    table_flat = tables.reshape(G * CODEBOOK, N_EMBD)

    mesh = plsc.VectorSubcoreMesh(core_axis_name="c", subcore_axis_name="s")

    @functools.partial(
        pl.kernel,
        out_type=jax.ShapeDtypeStruct((N, N_EMBD), jnp.float32),
        mesh=mesh,
        compiler_params=pltpu.CompilerParams(use_tc_tiling_on_sc=False),
        scratch_types=[
            pltpu.VMEM((NBUF, G, CHUNK), jnp.int32),
            pltpu.VMEM((NBUF, CHUNK, N_EMBD), jnp.float32),
            pltpu.SemaphoreType.DMA((NBUF,)),
            pltpu.SemaphoreType.DMA((NBUF,)),
        ],
    )
    def body(idx_hbm, tab_hbm, out_hbm, idx_v, acc_v, sem_idx, sem_acc):
        wid = lax.axis_index("s") * NC + lax.axis_index("c")
        cbase = wid * n_chunks

        def zero_acc(b):
            @pl.loop(0, CHUNK)
            def _(r):
                for k in range(N_EMBD // 16):
                    acc_v[b, r, pl.ds(k * 16, 16)] = jnp.zeros(
                        (16,), jnp.float32)

        def fire_gathers(b):
            for g in range(G):
                pltpu.async_copy(tab_hbm.at[idx_v.at[b, g]], acc_v.at[b],
                                 sem_acc.at[b], add=True)

        def drain_gathers(b):
            for g in range(G):
                pltpu.make_async_copy(tab_hbm.at[idx_v.at[b, g]],
                                      acc_v.at[b], sem_acc.at[b]).wait()

        def copy_out(b, j):
            pltpu.sync_copy(acc_v.at[b],
                            out_hbm.at[pl.ds((cbase + j) * CHUNK, CHUNK)])

        for b in range(NBUF):
            zero_acc(b)
            pltpu.sync_copy(idx_hbm.at[cbase + b], idx_v.at[b])
            fire_gathers(b)

        @pl.loop(0, n_chunks - NBUF, step=NBUF)
        def _(jj):
            for b in range(NBUF):
                j = jj + b
                drain_gathers(b)
                idx_cp = pltpu.async_copy(idx_hbm.at[cbase + j + NBUF],
                                          idx_v.at[b], sem_idx.at[b])
                copy_out(b, j)
                zero_acc(b)
                idx_cp.wait()
                fire_gathers(b)

        for b in range(NBUF):
            drain_gathers(b)
            copy_out(b, n_chunks - NBUF + b)

    out = body(idx_r, table_flat)
    return out.reshape(B, T, N_EMBD)
